# baseline (speedup 1.0000x reference)
import functools

import jax
import jax.numpy as jnp
from jax import lax
from jax.experimental import pallas as pl
from jax.experimental.pallas import tpu as pltpu
from jax.experimental.pallas import tpu_sc as plsc

_HID = 1024
_INTER = 4096
_E = 64
_NR = 32
_NF = 32
_T = 2048

_CB = 256
_NB_BASE = _INTER // _CB
_NB_AD = (_E * _NR) // _CB
_NSTEPS = _NB_BASE + 2 * _NB_AD

_NC = 2
_NS = 16
_TPW = _T // (_NC * _NS)
_ROWS_PW = _T // (_NC * _NS)


def _sc_sort_body(idx_hbm, scales_hbm, x_hbm,
                  inv_hbm, pos_hbm, idxs_hbm, sr_hbm, sf_hbm, xs_hbm,
                  idx_v, tok_v, pos_v, post_v, idxs_v, sr_v, sf_v, scales_v,
                  allh_v, cnt_v, off_v, buf_v, buf2_v, myinv_v, rows_v,
                  sh_hist, sh_inv, sh_idxs, sh_sr, sh_sf,
                  sem):
    core = lax.axis_index("c")
    sid = lax.axis_index("s")
    base = sid * 128
    l16 = lax.iota(jnp.int32, 16)

    pltpu.sync_copy(idx_hbm.at[pl.ds(base, 128)], idx_v)
    pltpu.sync_copy(scales_hbm, scales_v)

    def runs(c):
        k16 = idx_v[pl.ds(16 * c, 16)]
        ks, vs = plsc.sort_key_val(k16, l16)
        buf_v[pl.ds(0, 16)] = jnp.full((16,), -1, jnp.int32)
        buf_v[pl.ds(1, 16)] = ks
        prev = buf_v[pl.ds(0, 16)]
        buf2_v[pl.ds(1, 16)] = jnp.full((16,), -2, jnp.int32)
        buf2_v[pl.ds(0, 16)] = ks
        nxt = buf2_v[pl.ds(1, 16)]
        is_new = ks != prev
        last = ks != nxt
        run_start = plsc.cummax(l16, mask=is_new)
        rank = l16 - run_start
        return ks, vs, rank, last

    for k in range(_E // 16):
        cnt_v[pl.ds(16 * k, 16)] = jnp.zeros((16,), jnp.int32)
    for c in range(8):
        ks, vs, rank, last = runs(c)
        cur = plsc.load_gather(cnt_v, [ks])
        plsc.store_scatter(cnt_v, [ks], cur + rank + 1, mask=last)

    pltpu.sync_copy(cnt_v, sh_hist.at[sid])
    plsc.subcore_barrier()
    pltpu.sync_copy(sh_hist, allh_v)

    sid16 = jnp.full((16,), 0, jnp.int32) + sid
    carry = jnp.int32(0)
    for k in range(_E // 16):
        tot16 = jnp.zeros((16,), jnp.int32)
        mine16 = jnp.zeros((16,), jnp.int32)
        for w in range(_NS):
            row = allh_v[w, pl.ds(16 * k, 16)]
            tot16 = tot16 + row
            wlt = jnp.full((16,), w, jnp.int32) < sid16
            mine16 = mine16 + jnp.where(wlt, row, 0)
        gbase16 = carry + plsc.cumsum(tot16) - tot16
        off_v[pl.ds(16 * k, 16)] = gbase16 + mine16
        carry = carry + jnp.sum(tot16)

    for c in range(8):
        ks, vs, rank, last = runs(c)
        offs = plsc.load_gather(off_v, [ks])
        pos16 = offs + rank
        plsc.store_scatter(off_v, [ks], pos16 + 1, mask=last)
        pos_v[pl.ds(16 * c, 16)] = pos16
        tok_v[pl.ds(16 * c, 16)] = base + 16 * c + vs
        idxs_v[pl.ds(16 * c, 16)] = ks
        sr_v[pl.ds(16 * c, 16)] = plsc.load_gather(scales_v, [2 * ks])
        sf_v[pl.ds(16 * c, 16)] = plsc.load_gather(scales_v, [2 * ks + 1])
        plsc.store_scatter(post_v, [16 * c + vs], pos16)

    pltpu.async_copy(tok_v, sh_inv.at[pos_v], sem).wait()
    pltpu.async_copy(idxs_v, sh_idxs.at[pos_v], sem).wait()
    pltpu.async_copy(sr_v, sh_sr.at[pos_v], sem).wait()
    pltpu.async_copy(sf_v, sh_sf.at[pos_v], sem).wait()
    plsc.subcore_barrier()

    @pl.when(core == 0)
    def _():
        pltpu.sync_copy(sh_inv.at[pl.ds(base, 128)],
                        inv_hbm.at[pl.ds(base, 128)])
        pltpu.sync_copy(sh_idxs.at[pl.ds(base, 128)],
                        idxs_hbm.at[pl.ds(base, 128)])
        pltpu.sync_copy(sh_sr.at[pl.ds(base, 128)],
                        sr_hbm.at[pl.ds(base, 128)])
        pltpu.sync_copy(sh_sf.at[pl.ds(base, 128)],
                        sf_hbm.at[pl.ds(base, 128)])
        pltpu.sync_copy(post_v, pos_hbm.at[pl.ds(base, 128)])

    row0 = core * 1024 + sid * 64
    pltpu.sync_copy(sh_inv.at[pl.ds(row0, 64)], myinv_v)
    for j in range(4):
        pltpu.async_copy(x_hbm.at[myinv_v.at[pl.ds(16 * j, 16)]],
                         rows_v, sem).wait()
        pltpu.sync_copy(rows_v, xs_hbm.at[pl.ds(row0 + 16 * j, 16)])


def _sc_sort(idx, scales_flat, x):
    mesh = plsc.VectorSubcoreMesh(core_axis_name="c", subcore_axis_name="s")
    f = pl.kernel(
        _sc_sort_body,
        mesh=mesh,
        out_type=(
            jax.ShapeDtypeStruct((_T,), jnp.int32),
            jax.ShapeDtypeStruct((_T,), jnp.int32),
            jax.ShapeDtypeStruct((_T,), jnp.int32),
            jax.ShapeDtypeStruct((_T,), jnp.float32),
            jax.ShapeDtypeStruct((_T,), jnp.float32),
            jax.ShapeDtypeStruct((_T, _HID), jnp.float32),
        ),
        scratch_types=[
            pltpu.VMEM((128,), jnp.int32),
            pltpu.VMEM((128,), jnp.int32),
            pltpu.VMEM((128,), jnp.int32),
            pltpu.VMEM((128,), jnp.int32),
            pltpu.VMEM((128,), jnp.int32),
            pltpu.VMEM((128,), jnp.float32),
            pltpu.VMEM((128,), jnp.float32),
            pltpu.VMEM((128,), jnp.float32),
            pltpu.VMEM((_NS, _E), jnp.int32),
            pltpu.VMEM((_E,), jnp.int32),
            pltpu.VMEM((_E,), jnp.int32),
            pltpu.VMEM((32,), jnp.int32),
            pltpu.VMEM((32,), jnp.int32),
            pltpu.VMEM((64,), jnp.int32),
            pltpu.VMEM((16, _HID), jnp.float32),
            pltpu.VMEM_SHARED((_NS, _E), jnp.int32),
            pltpu.VMEM_SHARED((_T,), jnp.int32),
            pltpu.VMEM_SHARED((_T,), jnp.int32),
            pltpu.VMEM_SHARED((_T,), jnp.float32),
            pltpu.VMEM_SHARED((_T,), jnp.float32),
            pltpu.SemaphoreType.DMA,
        ],
        compiler_params=pltpu.CompilerParams(needs_layout_passes=False),
    )
    return f(idx, scales_flat, x)


def _sc_unsort_body(pos_hbm, outs_hbm, out_hbm, pos_v, rows_v, sem):
    core = lax.axis_index("c")
    sid = lax.axis_index("s")
    row0 = core * 1024 + sid * 64
    pltpu.sync_copy(pos_hbm.at[pl.ds(row0, 64)], pos_v)
    for j in range(4):
        pltpu.async_copy(outs_hbm.at[pos_v.at[pl.ds(16 * j, 16)]],
                         rows_v, sem).wait()
        pltpu.sync_copy(rows_v, out_hbm.at[pl.ds(row0 + 16 * j, 16)])


def _sc_unsort(pos, out_sorted):
    mesh = plsc.VectorSubcoreMesh(core_axis_name="c", subcore_axis_name="s")
    f = pl.kernel(
        _sc_unsort_body,
        mesh=mesh,
        out_type=jax.ShapeDtypeStruct((_T, _HID), jnp.float32),
        scratch_types=[
            pltpu.VMEM((64,), jnp.int32),
            pltpu.VMEM((16, _HID), jnp.float32),
            pltpu.SemaphoreType.DMA,
        ],
    )
    return f(pos, out_sorted)


def _mlp_body(idx_ref, sr_ref, sf_ref, x_ref, bg_ref, bu_ref, rg_ref, ru_ref,
              fg_ref, fu_ref, bd_ref, rd_ref, fd_ref, out_ref):
    c = pl.program_id(0)
    x = x_ref[...]

    def swiglu(g_w, u_w):
        dn = (((1,), (1,)), ((), ()))
        g = lax.dot_general(x, g_w.astype(jnp.bfloat16), dn,
                            preferred_element_type=jnp.float32)
        u = lax.dot_general(x, u_w.astype(jnp.bfloat16), dn,
                            preferred_element_type=jnp.float32)
        sig = 1.0 / (1.0 + jnp.exp(-g))
        return (g * sig) * u

    def accum(contrib):
        @pl.when(c == 0)
        def _():
            out_ref[...] = contrib

        @pl.when(c > 0)
        def _():
            out_ref[...] += contrib

    def adapter_mask(h, block_in_adapter, s_ref):
        col = block_in_adapter * _CB + lax.broadcasted_iota(
            jnp.int32, (_T, _CB), 1)
        ecol = col // _NR
        idxv = idx_ref[...]
        sel = (ecol == idxv)
        return jnp.where(sel, h * s_ref[...], 0.0)

    @pl.when(c < _NB_BASE)
    def _():
        h = swiglu(bg_ref[...], bu_ref[...])
        dn = (((1,), (1,)), ((), ()))
        contrib = lax.dot_general(
            h.astype(jnp.bfloat16), bd_ref[...].astype(jnp.bfloat16), dn,
            preferred_element_type=jnp.float32)
        accum(contrib)

    @pl.when((c >= _NB_BASE) & (c < _NB_BASE + _NB_AD))
    def _():
        h = swiglu(rg_ref[...], ru_ref[...])
        h = adapter_mask(h, c - _NB_BASE, sr_ref)
        contrib = jnp.dot(h.astype(jnp.bfloat16), rd_ref[...],
                          preferred_element_type=jnp.float32)
        accum(contrib)

    @pl.when(c >= _NB_BASE + _NB_AD)
    def _():
        h = swiglu(fg_ref[...], fu_ref[...])
        h = adapter_mask(h, c - _NB_BASE - _NB_AD, sf_ref)
        contrib = jnp.dot(h.astype(jnp.bfloat16), fd_ref[...],
                          preferred_element_type=jnp.float32)
        accum(contrib)


def kernel(x, token_lora_indices, base_gate_w, base_up_w, base_down_w,
           retain_gate_stacked, retain_up_stacked, retain_down_stacked,
           forget_gate_stacked, forget_up_stacked, forget_down_stacked,
           scales):
    idx0 = jnp.maximum(token_lora_indices, 0)
    inv, pos, idxs, sr, sf, x_sorted = _sc_sort(
        idx0, scales.reshape(_E * 2), x)

    idxs2 = idxs.reshape(_T, 1)
    sr2 = sr.reshape(_T, 1)
    sf2 = sf.reshape(_T, 1)
    xb = x_sorted.astype(jnp.bfloat16)

    rg = retain_gate_stacked.reshape(_E * _NR, _HID)
    ru = retain_up_stacked.reshape(_E * _NR, _HID)
    fg = forget_gate_stacked.reshape(_E * _NF, _HID)
    fu = forget_up_stacked.reshape(_E * _NF, _HID)
    rd = retain_down_stacked[:, 0].transpose(0, 2, 1).reshape(
        _E * _NR, _HID).astype(jnp.bfloat16)
    fd = forget_down_stacked[:, 0].transpose(0, 2, 1).reshape(
        _E * _NF, _HID).astype(jnp.bfloat16)

    nb = _NB_BASE
    na = _NB_AD

    def clamp(lo, hi):
        return lambda c: (jnp.clip(c - lo, 0, hi - 1), 0)

    grid_spec = dict(
        grid=(_NSTEPS,),
        in_specs=[
            pl.BlockSpec((_T, 1), lambda c: (0, 0)),
            pl.BlockSpec((_T, 1), lambda c: (0, 0)),
            pl.BlockSpec((_T, 1), lambda c: (0, 0)),
            pl.BlockSpec((_T, _HID), lambda c: (0, 0)),
            pl.BlockSpec((_CB, _HID), clamp(0, nb)),
            pl.BlockSpec((_CB, _HID), clamp(0, nb)),
            pl.BlockSpec((_CB, _HID), clamp(nb, na)),
            pl.BlockSpec((_CB, _HID), clamp(nb, na)),
            pl.BlockSpec((_CB, _HID), clamp(nb + na, na)),
            pl.BlockSpec((_CB, _HID), clamp(nb + na, na)),
            pl.BlockSpec((_HID, _CB),
                         lambda c: (0, jnp.clip(c, 0, nb - 1))),
            pl.BlockSpec((_CB, _HID), clamp(nb, na)),
            pl.BlockSpec((_CB, _HID), clamp(nb + na, na)),
        ],
        out_specs=pl.BlockSpec((_T, _HID), lambda c: (0, 0)),
    )

    out_sorted = pl.pallas_call(
        _mlp_body,
        **grid_spec,
        out_shape=jax.ShapeDtypeStruct((_T, _HID), jnp.float32),
        compiler_params=pltpu.CompilerParams(
            dimension_semantics=("arbitrary",)),
    )(idxs2, sr2, sf2, xb, base_gate_w, base_up_w, rg, ru, fg, fu,
      base_down_w, rd, fd)

    return _sc_unsort(pos, out_sorted)
